# Initial kernel scaffold; baseline (speedup 1.0000x reference)
import jax
import jax.numpy as jnp
from jax.experimental import pallas as pl


def kernel(feat, proposals, params):
    raise NotImplementedError("write your pallas kernel here")


# speedup vs baseline: 3.2361x; 3.2361x over previous
import functools
import jax
import jax.numpy as jnp
from jax.experimental import pallas as pl
from jax.experimental.pallas import tpu as pltpu

_NC = 81
_IMG = 800.0
_SCALE = 0.25
_OUT = 7
_REP = 1024
_C = 256
_DET = 500
_PRE = 1000
_SCORE_TH = 0.05
_NMS_TH = 0.5
_CLIP = 4.135166556742356

_KB = 1792
_KSTEPS = 7


def _head_body(x_ref, w1_ref, b1_ref, w2_ref, b2_ref, wc_ref, bc_ref,
               wr_ref, br_ref, cls_ref, reg_ref, acc_ref):
    k = pl.program_id(0)

    @pl.when(k == 0)
    def _():
        acc_ref[:] = jnp.zeros_like(acc_ref)

    acc_ref[:] += jnp.dot(x_ref[:], w1_ref[:],
                          preferred_element_type=jnp.float32)

    @pl.when(k == _KSTEPS - 1)
    def _():
        h1 = jnp.maximum(acc_ref[:] + b1_ref[:], 0.0)
        h2 = jnp.maximum(
            jnp.dot(h1, w2_ref[:], preferred_element_type=jnp.float32)
            + b2_ref[:], 0.0)
        cls_ref[:] = jnp.dot(h2, wc_ref[:],
                             preferred_element_type=jnp.float32) + bc_ref[:]
        reg_ref[:] = jnp.dot(h2, wr_ref[:],
                             preferred_element_type=jnp.float32) + br_ref[:]


@functools.partial(jax.jit, static_argnums=())
def _head_pallas(xp, w1, b1, w2, b2, wc, bc, wr, br):
    n = xp.shape[0]
    grid = (_KSTEPS,)
    return pl.pallas_call(
        _head_body,
        grid=grid,
        in_specs=[
            pl.BlockSpec((n, _KB), lambda k: (0, k)),
            pl.BlockSpec((_KB, _REP), lambda k: (k, 0)),
            pl.BlockSpec((1, _REP), lambda k: (0, 0)),
            pl.BlockSpec((_REP, _REP), lambda k: (0, 0)),
            pl.BlockSpec((1, _REP), lambda k: (0, 0)),
            pl.BlockSpec((_REP, 128), lambda k: (0, 0)),
            pl.BlockSpec((1, 128), lambda k: (0, 0)),
            pl.BlockSpec((_REP, 384), lambda k: (0, 0)),
            pl.BlockSpec((1, 384), lambda k: (0, 0)),
        ],
        out_specs=[
            pl.BlockSpec((n, 128), lambda k: (0, 0)),
            pl.BlockSpec((n, 384), lambda k: (0, 0)),
        ],
        out_shape=[
            jax.ShapeDtypeStruct((n, 128), jnp.float32),
            jax.ShapeDtypeStruct((n, 384), jnp.float32),
        ],
        scratch_shapes=[pltpu.VMEM((n, _REP), jnp.float32)],
        compiler_params=pltpu.CompilerParams(
            dimension_semantics=("arbitrary",)),
    )(xp, w1, b1, w2, b2, wc, bc, wr, br)


def _nms_body(ts_ref, sc0_ref, bx1_ref, by1_ref, bx2_ref, by2_ref, tl_ref,
              out_ref, sc_ref):
    sc_ref[:] = sc0_ref[:]
    off = tl_ref[:] * (_IMG + 1.0)
    ox1 = bx1_ref[:] + off
    oy1 = by1_ref[:] + off
    ox2 = bx2_ref[:] + off
    oy2 = by2_ref[:] + off
    area2 = (jnp.maximum(ox2 - ox1, 0.0) * jnp.maximum(oy2 - oy1, 0.0))
    idx = jax.lax.broadcasted_iota(jnp.int32, (8, 128), 0) * 128 + \
        jax.lax.broadcasted_iota(jnp.int32, (8, 128), 1)
    lane8 = jax.lax.broadcasted_iota(jnp.int32, (1, 8), 1)

    def step(i, _):
        sc = sc_ref[:]
        m = jnp.max(sc)
        j = jnp.min(jnp.where(sc == m, idx, jnp.int32(1 << 30)))
        selm = idx == j

        def pick(v):
            return jnp.sum(jnp.where(selm, v, 0.0))

        sx1 = pick(ox1)
        sy1 = pick(oy1)
        sx2 = pick(ox2)
        sy2 = pick(oy2)
        a1 = jnp.maximum(sx2 - sx1, 0.0) * jnp.maximum(sy2 - sy1, 0.0)
        inter = (jnp.maximum(jnp.minimum(sx2, ox2) - jnp.maximum(sx1, ox1), 0.0)
                 * jnp.maximum(jnp.minimum(sy2, oy2) - jnp.maximum(sy1, oy1),
                               0.0))
        iou = inter / (a1 + area2 - inter + 1e-9)
        sc = jnp.where(iou > _NMS_TH, -1e9, sc)
        sc_ref[:] = jnp.where(selm, -1e9, sc)
        val = jnp.where(m > 0.0, 1.0, 0.0)
        px1 = pick(bx1_ref[:]) * val
        py1 = pick(by1_ref[:]) * val
        px2 = pick(bx2_ref[:]) * val
        py2 = pick(by2_ref[:]) * val
        psc = pick(ts_ref[:]) * val
        row = jnp.where(lane8 == 0, px1,
              jnp.where(lane8 == 1, py1,
              jnp.where(lane8 == 2, px2,
              jnp.where(lane8 == 3, py2,
              jnp.where(lane8 == 4, psc, 0.0)))))
        out_ref[pl.ds(i, 1), :] = row
        return 0

    jax.lax.fori_loop(0, _DET, step, 0)


def _nms_pallas(ts, sc0, bx1, by1, bx2, by2, tl):
    return pl.pallas_call(
        _nms_body,
        out_shape=jax.ShapeDtypeStruct((512, 8), jnp.float32),
        scratch_shapes=[pltpu.VMEM((8, 128), jnp.float32)],
    )(ts, sc0, bx1, by1, bx2, by2, tl)


def _roi_align(feat, rois):
    fmap = feat[0]
    c, h, w = fmap.shape
    n = rois.shape[0]
    x1 = rois[:, 0] * _SCALE
    y1 = rois[:, 1] * _SCALE
    x2 = rois[:, 2] * _SCALE
    y2 = rois[:, 3] * _SCALE
    rw = jnp.maximum(x2 - x1, 1.0)
    rh = jnp.maximum(y2 - y1, 1.0)
    off = jnp.arange(_OUT, dtype=jnp.float32) + 0.5
    px = x1[:, None] + off[None, :] * (rw / _OUT)[:, None]
    py = y1[:, None] + off[None, :] * (rh / _OUT)[:, None]
    gx = jnp.broadcast_to(px[:, None, :], (n, _OUT, _OUT))
    gy = jnp.broadcast_to(py[:, :, None], (n, _OUT, _OUT))
    x0 = jnp.floor(gx)
    y0 = jnp.floor(gy)
    lx = gx - x0
    ly = gy - y0
    x0i = jnp.clip(x0, 0, w - 1).astype(jnp.int32)
    x1i = jnp.clip(x0 + 1, 0, w - 1).astype(jnp.int32)
    y0i = jnp.clip(y0, 0, h - 1).astype(jnp.int32)
    y1i = jnp.clip(y0 + 1, 0, h - 1).astype(jnp.int32)
    v00 = fmap[:, y0i, x0i]
    v01 = fmap[:, y0i, x1i]
    v10 = fmap[:, y1i, x0i]
    v11 = fmap[:, y1i, x1i]
    out = (v00 * ((1 - ly) * (1 - lx))[None]
           + v01 * ((1 - ly) * lx)[None]
           + v10 * (ly * (1 - lx))[None]
           + v11 * (ly * lx)[None])
    return jnp.transpose(out, (1, 0, 2, 3))


def _decode(deltas, boxes):
    widths = boxes[:, 2] - boxes[:, 0]
    heights = boxes[:, 3] - boxes[:, 1]
    ctrx = boxes[:, 0] + 0.5 * widths
    ctry = boxes[:, 1] + 0.5 * heights
    d = deltas.reshape(boxes.shape[0], _NC, 4)
    dx = d[..., 0] / 10.0
    dy = d[..., 1] / 10.0
    dw = jnp.minimum(d[..., 2] / 5.0, _CLIP)
    dh = jnp.minimum(d[..., 3] / 5.0, _CLIP)
    pcx = dx * widths[:, None] + ctrx[:, None]
    pcy = dy * heights[:, None] + ctry[:, None]
    pw = jnp.exp(dw) * widths[:, None]
    ph = jnp.exp(dh) * heights[:, None]
    return jnp.stack([pcx - 0.5 * pw, pcy - 0.5 * ph,
                      pcx + 0.5 * pw, pcy + 0.5 * ph], axis=-1)


def _clip_boxes(b):
    return jnp.stack([jnp.clip(b[:, 0], 0.0, _IMG), jnp.clip(b[:, 1], 0.0, _IMG),
                      jnp.clip(b[:, 2], 0.0, _IMG), jnp.clip(b[:, 3], 0.0, _IMG)],
                     axis=1)


def _stage(feat, props, params, s):
    pooled = _roi_align(feat, props)
    x = pooled.reshape(props.shape[0], -1)
    xp = jnp.pad(x, ((0, 1024 - x.shape[0]), (0, 0)))
    wc = jnp.pad(params['cls_w_%d' % s], ((0, 0), (0, 128 - _NC)))
    bc = jnp.pad(params['cls_b_%d' % s], (0, 128 - _NC)).reshape(1, 128)
    wr = jnp.pad(params['reg_w_%d' % s], ((0, 0), (0, 384 - 4 * _NC)))
    br = jnp.pad(params['reg_b_%d' % s], (0, 384 - 4 * _NC)).reshape(1, 384)
    cls_p, reg_p = _head_pallas(
        xp, params['fc1_w_%d' % s], params['fc1_b_%d' % s].reshape(1, _REP),
        params['fc2_w_%d' % s], params['fc2_b_%d' % s].reshape(1, _REP),
        wc, bc, wr, br)
    n = props.shape[0]
    return cls_p[:n, :_NC], reg_p[:n, :4 * _NC]


def kernel(feat, proposals, params):
    props = proposals
    n = props.shape[0]
    all_cls = []
    reg = None
    for s in range(3):
        cls, reg = _stage(feat, props, params, s)
        all_cls.append(cls)
        if s < 2:
            dec = _decode(reg, props)
            refined = dec[:, 1:, :].mean(axis=1)
            props = _clip_boxes(refined)
    scores = sum(jax.nn.softmax(c, axis=-1) for c in all_cls) / 3.0
    boxes = _decode(reg, props)
    boxes = _clip_boxes(boxes.reshape(-1, 4)).reshape(n, _NC, 4)
    fb = boxes[:, 1:, :].reshape(-1, 4)
    fs = scores[:, 1:].reshape(-1)
    fl = jnp.broadcast_to(jnp.arange(1, _NC)[None, :], (n, _NC - 1)).reshape(-1)
    ws_ = fb[:, 2] - fb[:, 0]
    hs_ = fb[:, 3] - fb[:, 1]
    valid = (fs > _SCORE_TH) & (ws_ > 1e-2) & (hs_ > 1e-2)
    fsm = jnp.where(valid, fs, -1.0)
    top_s, top_i = jax.lax.top_k(fsm, _PRE)
    tb = fb[top_i]
    tl = fl[top_i].astype(jnp.float32)
    ts = fs[top_i]

    def pad8(v, fill):
        return jnp.pad(v, (0, 1024 - _PRE),
                       constant_values=fill).reshape(8, 128)

    out = _nms_pallas(pad8(ts, 0.0), pad8(top_s, -1e9),
                      pad8(tb[:, 0], 0.0), pad8(tb[:, 1], 0.0),
                      pad8(tb[:, 2], 0.0), pad8(tb[:, 3], 0.0),
                      pad8(tl, 0.0))
    return out[:_DET, :5]
